# Initial kernel scaffold; baseline (speedup 1.0000x reference)
import jax
import jax.numpy as jnp
from jax.experimental import pallas as pl


def kernel(x_view0, x_view1, x_view2, edge_index_view0, edge_index_view1, edge_index_view2, batch, edge_attr_view0, edge_attr_view1, edge_attr_view2, istrain, W_body, b_body, W_gate, b_gate, W_noise, b_noise, W_root, b_root, W_msg, b_msg):
    raise NotImplementedError("write your pallas kernel here")


# speedup vs baseline: 1.4504x; 1.4504x over previous
import jax
import jax.numpy as jnp
from jax.experimental import pallas as pl

_N = 10000
_E = 320000
_D = 128
_ED = 16
_NEXP = 8
_K = 2
_V = 3
_G = 64
_DG = 2048
_COEF = 0.01

_BE = 2000
_BN = 2000


def _gating_body(gf_ref, wb_ref, bb_ref, wg_ref, bg_ref, h_ref, logits_ref):
    h = jnp.maximum(
        jnp.dot(gf_ref[...], wb_ref[...], preferred_element_type=jnp.float32)
        + bb_ref[...], 0.0)
    h_ref[...] = h
    logits_ref[...] = (
        jnp.dot(h, wg_ref[...], preferred_element_type=jnp.float32) + bg_ref[...])


def _msg_body(xs_ref, ea_ref, wx_ref, we_ref, b_ref, out_ref):
    acc = jnp.dot(xs_ref[...], wx_ref[...], preferred_element_type=jnp.float32)
    acc = acc + jnp.dot(ea_ref[...], we_ref[...], preferred_element_type=jnp.float32)
    out_ref[...] = jnp.maximum(acc + b_ref[...], 0.0)


def _combine_body(x_ref, agg_ref, gates_ref, wr_ref, br_ref, out_ref):
    h = (jnp.dot(x_ref[...], wr_ref[...], preferred_element_type=jnp.float32)
         + br_ref[...] + agg_ref[...])
    h = jnp.maximum(h, 0.0)
    g = gates_ref[...]
    acc = g[:, 0:1] * h[:, 0:_D]
    for i in range(1, _NEXP):
        acc = acc + g[:, i:i + 1] * h[:, i * _D:(i + 1) * _D]
    out_ref[...] = acc


def _cv_squared(x):
    x = x.astype(jnp.float32)
    return jnp.var(x, ddof=1) / (jnp.mean(x) ** 2 + 1e-10)


def kernel(x_view0, x_view1, x_view2, edge_index_view0, edge_index_view1,
           edge_index_view2, batch, edge_attr_view0, edge_attr_view1,
           edge_attr_view2, istrain, W_body, b_body, W_gate, b_gate,
           W_noise, b_noise, W_root, b_root, W_msg, b_msg):
    x_views = [x_view0, x_view1, x_view2]
    ei_views = [edge_index_view0, edge_index_view1, edge_index_view2]
    ea_views = [edge_attr_view0, edge_attr_view1, edge_attr_view2]

    ones_n = jnp.ones((_N,), jnp.float32)
    cnt = jnp.maximum(jax.ops.segment_sum(ones_n, batch, num_segments=_G), 1.0)
    gfs = [jax.ops.segment_sum(xv, batch, num_segments=_G) / cnt[:, None]
           for xv in x_views]
    gf_all = jnp.concatenate(gfs, axis=0)

    h_all, logits_all = pl.pallas_call(
        _gating_body,
        out_shape=(
            jax.ShapeDtypeStruct((_V * _G, _DG), jnp.float32),
            jax.ShapeDtypeStruct((_V * _G, _NEXP), jnp.float32),
        ),
    )(gf_all, W_body, b_body.reshape(1, _DG), W_gate, b_gate.reshape(1, _NEXP))

    subcls, tk_gates, tk_idx, gfl_list, load_list, subw = [], [], [], [], [], []
    for v in range(_V):
        logits = logits_all[v * _G:(v + 1) * _G]
        subcls.append(h_all[v * _G:(v + 1) * _G])
        top_logits, top_indices = jax.lax.top_k(logits, _K)
        gates_v = jax.nn.softmax(top_logits, axis=1)
        gfl = jnp.zeros_like(logits).at[jnp.arange(_G)[:, None], top_indices].set(1.0)
        load_list.append(gfl.sum(axis=0))
        gfl_list.append(gfl)
        tk_gates.append(gates_v)
        tk_idx.append(top_indices)
        norms = jnp.linalg.norm(logits, axis=1)
        subw.append((norms - norms.min()) / (norms.max() - norms.min() + 1e-6) + 0.1)

    importance = jnp.concatenate(gfl_list, axis=0).sum(axis=0)
    all_load = jnp.stack(load_list).sum(axis=0)
    loss = (_cv_squared(importance) + _cv_squared(all_load)) * _COEF

    Wx = jnp.transpose(W_msg[:, :_D, :], (1, 0, 2)).reshape(_D, _NEXP * _D)
    We = jnp.transpose(W_msg[:, _D:, :], (1, 0, 2)).reshape(_ED, _NEXP * _D)
    bm = b_msg.reshape(1, _NEXP * _D)
    Wr = jnp.transpose(W_root, (1, 0, 2)).reshape(_D, _NEXP * _D)
    br = b_root.reshape(1, _NEXP * _D)

    msg_call = pl.pallas_call(
        _msg_body,
        grid=(_E // _BE,),
        in_specs=[
            pl.BlockSpec((_BE, _D), lambda i: (i, 0)),
            pl.BlockSpec((_BE, _ED), lambda i: (i, 0)),
            pl.BlockSpec((_D, _NEXP * _D), lambda i: (0, 0)),
            pl.BlockSpec((_ED, _NEXP * _D), lambda i: (0, 0)),
            pl.BlockSpec((1, _NEXP * _D), lambda i: (0, 0)),
        ],
        out_specs=pl.BlockSpec((_BE, _NEXP * _D), lambda i: (i, 0)),
        out_shape=jax.ShapeDtypeStruct((_E, _NEXP * _D), jnp.float32),
    )

    combine_call = pl.pallas_call(
        _combine_body,
        grid=(_N // _BN,),
        in_specs=[
            pl.BlockSpec((_BN, _D), lambda i: (i, 0)),
            pl.BlockSpec((_BN, _NEXP * _D), lambda i: (i, 0)),
            pl.BlockSpec((_BN, _NEXP), lambda i: (i, 0)),
            pl.BlockSpec((_D, _NEXP * _D), lambda i: (0, 0)),
            pl.BlockSpec((1, _NEXP * _D), lambda i: (0, 0)),
        ],
        out_specs=pl.BlockSpec((_BN, _D), lambda i: (i, 0)),
        out_shape=jax.ShapeDtypeStruct((_N, _D), jnp.float32),
    )

    final_nodes = []
    ones_e = jnp.ones((_E,), jnp.float32)
    for v in range(_V):
        src = ei_views[v][0]
        dst = ei_views[v][1]
        xs = jnp.take(x_views[v], src, axis=0)
        m = msg_call(xs, ea_views[v], Wx, We, bm)
        agg = jax.ops.segment_sum(m, dst, num_segments=_N)
        deg = jnp.maximum(jax.ops.segment_sum(ones_e, dst, num_segments=_N), 1.0)
        agg = agg / deg[:, None]
        gates_graph = (jnp.zeros((_G, _NEXP), jnp.float32)
                       .at[jnp.arange(_G)[:, None], tk_idx[v]].set(tk_gates[v]))
        node_gates = jnp.take(gates_graph, batch, axis=0)
        final_nodes.append(combine_call(x_views[v], agg, node_gates, Wr, br))

    graph_outs = [jax.ops.segment_sum(xn, batch, num_segments=_G) / cnt[:, None]
                  for xn in final_nodes]
    x_graph = jnp.stack(graph_outs, axis=0)
    return (final_nodes[0], final_nodes[1], final_nodes[2], x_graph, loss,
            subcls[0], subcls[1], subcls[2], subw[0], subw[1], subw[2])


# speedup vs baseline: 1.5118x; 1.0423x over previous
import jax
import jax.numpy as jnp
from jax.experimental import pallas as pl

_N = 10000
_E = 320000
_D = 128
_ED = 16
_NEXP = 8
_K = 2
_V = 3
_G = 64
_DG = 2048
_COEF = 0.01

_BE = 2000
_BN = 2000


def _gating_body(gf_ref, wb_ref, bb_ref, wg_ref, bg_ref, h_ref, logits_ref):
    h = jnp.maximum(
        jnp.dot(gf_ref[...], wb_ref[...], preferred_element_type=jnp.float32)
        + bb_ref[...], 0.0)
    h_ref[...] = h
    logits_ref[...] = (
        jnp.dot(h, wg_ref[...], preferred_element_type=jnp.float32) + bg_ref[...])


def _msg_body(xs_ref, ea_ref, wx_ref, we_ref, b_ref, out_ref):
    acc = jnp.dot(xs_ref[...], wx_ref[...], preferred_element_type=jnp.float32)
    acc = acc + jnp.dot(ea_ref[...], we_ref[...], preferred_element_type=jnp.float32)
    out_ref[...] = jnp.maximum(acc + b_ref[...], 0.0).astype(jnp.bfloat16)


def _combine_body(x_ref, agg_ref, gates_ref, wr_ref, br_ref, out_ref):
    h = (jnp.dot(x_ref[...], wr_ref[...], preferred_element_type=jnp.float32)
         + br_ref[...] + agg_ref[...])
    h = jnp.maximum(h, 0.0)
    g = gates_ref[...]
    acc = g[:, 0:1] * h[:, 0:_D]
    for i in range(1, _NEXP):
        acc = acc + g[:, i:i + 1] * h[:, i * _D:(i + 1) * _D]
    out_ref[...] = acc


def _cv_squared(x):
    x = x.astype(jnp.float32)
    return jnp.var(x, ddof=1) / (jnp.mean(x) ** 2 + 1e-10)


def kernel(x_view0, x_view1, x_view2, edge_index_view0, edge_index_view1,
           edge_index_view2, batch, edge_attr_view0, edge_attr_view1,
           edge_attr_view2, istrain, W_body, b_body, W_gate, b_gate,
           W_noise, b_noise, W_root, b_root, W_msg, b_msg):
    x_views = [x_view0, x_view1, x_view2]
    ei_views = [edge_index_view0, edge_index_view1, edge_index_view2]
    ea_views = [edge_attr_view0, edge_attr_view1, edge_attr_view2]

    ones_n = jnp.ones((_N,), jnp.float32)
    cnt = jnp.maximum(jax.ops.segment_sum(ones_n, batch, num_segments=_G), 1.0)
    gfs = [jax.ops.segment_sum(xv, batch, num_segments=_G) / cnt[:, None]
           for xv in x_views]
    gf_all = jnp.concatenate(gfs, axis=0)

    h_all, logits_all = pl.pallas_call(
        _gating_body,
        out_shape=(
            jax.ShapeDtypeStruct((_V * _G, _DG), jnp.float32),
            jax.ShapeDtypeStruct((_V * _G, _NEXP), jnp.float32),
        ),
    )(gf_all, W_body, b_body.reshape(1, _DG), W_gate, b_gate.reshape(1, _NEXP))

    subcls, tk_gates, tk_idx, gfl_list, load_list, subw = [], [], [], [], [], []
    for v in range(_V):
        logits = logits_all[v * _G:(v + 1) * _G]
        subcls.append(h_all[v * _G:(v + 1) * _G])
        top_logits, top_indices = jax.lax.top_k(logits, _K)
        gates_v = jax.nn.softmax(top_logits, axis=1)
        gfl = jnp.zeros_like(logits).at[jnp.arange(_G)[:, None], top_indices].set(1.0)
        load_list.append(gfl.sum(axis=0))
        gfl_list.append(gfl)
        tk_gates.append(gates_v)
        tk_idx.append(top_indices)
        norms = jnp.linalg.norm(logits, axis=1)
        subw.append((norms - norms.min()) / (norms.max() - norms.min() + 1e-6) + 0.1)

    importance = jnp.concatenate(gfl_list, axis=0).sum(axis=0)
    all_load = jnp.stack(load_list).sum(axis=0)
    loss = (_cv_squared(importance) + _cv_squared(all_load)) * _COEF

    Wx = jnp.transpose(W_msg[:, :_D, :], (1, 0, 2)).reshape(_D, _NEXP * _D)
    We = jnp.transpose(W_msg[:, _D:, :], (1, 0, 2)).reshape(_ED, _NEXP * _D)
    bm = b_msg.reshape(1, _NEXP * _D)
    Wr = jnp.transpose(W_root, (1, 0, 2)).reshape(_D, _NEXP * _D)
    br = b_root.reshape(1, _NEXP * _D)

    msg_call = pl.pallas_call(
        _msg_body,
        grid=(_E // _BE,),
        in_specs=[
            pl.BlockSpec((_BE, _D), lambda i: (i, 0)),
            pl.BlockSpec((_BE, _ED), lambda i: (i, 0)),
            pl.BlockSpec((_D, _NEXP * _D), lambda i: (0, 0)),
            pl.BlockSpec((_ED, _NEXP * _D), lambda i: (0, 0)),
            pl.BlockSpec((1, _NEXP * _D), lambda i: (0, 0)),
        ],
        out_specs=pl.BlockSpec((_BE, _NEXP * _D), lambda i: (i, 0)),
        out_shape=jax.ShapeDtypeStruct((_E, _NEXP * _D), jnp.bfloat16),
    )

    combine_call = pl.pallas_call(
        _combine_body,
        grid=(_N // _BN,),
        in_specs=[
            pl.BlockSpec((_BN, _D), lambda i: (i, 0)),
            pl.BlockSpec((_BN, _NEXP * _D), lambda i: (i, 0)),
            pl.BlockSpec((_BN, _NEXP), lambda i: (i, 0)),
            pl.BlockSpec((_D, _NEXP * _D), lambda i: (0, 0)),
            pl.BlockSpec((1, _NEXP * _D), lambda i: (0, 0)),
        ],
        out_specs=pl.BlockSpec((_BN, _D), lambda i: (i, 0)),
        out_shape=jax.ShapeDtypeStruct((_N, _D), jnp.float32),
    )

    final_nodes = []
    ones_e = jnp.ones((_E,), jnp.float32)
    for v in range(_V):
        src = ei_views[v][0]
        dst = ei_views[v][1]
        xs = jnp.take(x_views[v], src, axis=0)
        m = msg_call(xs, ea_views[v], Wx, We, bm)
        agg = jax.ops.segment_sum(m.astype(jnp.float32), dst, num_segments=_N)
        deg = jnp.maximum(jax.ops.segment_sum(ones_e, dst, num_segments=_N), 1.0)
        agg = agg / deg[:, None]
        gates_graph = (jnp.zeros((_G, _NEXP), jnp.float32)
                       .at[jnp.arange(_G)[:, None], tk_idx[v]].set(tk_gates[v]))
        node_gates = jnp.take(gates_graph, batch, axis=0)
        final_nodes.append(combine_call(x_views[v], agg, node_gates, Wr, br))

    graph_outs = [jax.ops.segment_sum(xn, batch, num_segments=_G) / cnt[:, None]
                  for xn in final_nodes]
    x_graph = jnp.stack(graph_outs, axis=0)
    return (final_nodes[0], final_nodes[1], final_nodes[2], x_graph, loss,
            subcls[0], subcls[1], subcls[2], subw[0], subw[1], subw[2])
